# baseline (speedup 1.0000x reference)
import jax
import jax.numpy as jnp
from jax.experimental import pallas as pl
from jax.experimental.pallas import tpu as pltpu

B, D, H, R, K, NL = 4096, 384, 64, 8, 2, 64
RH = R * H
AUX_COEF = 0.05
TILE = 1024
GRID = B // TILE

_NEG = -3.0e38


def _dot(a, b):
    return jnp.dot(a, b, preferred_element_type=jnp.float32)


def _moe_kernel(x1_ref, x2_ref, leT_ref, gw_ref, gbc_ref, uc_ref, ub_ref,
                vc_ref, vbc_ref, g_ref, fz_ref, ebc_ref, tri_ref, f_ref,
                ones_ref, rand_ref, sel_ref, logp_ref, aux_ref,
                m_ref, accp_ref, accm_ref):
    i = pl.program_id(0)
    x1 = x1_ref[...]
    x2 = x2_ref[...]

    @pl.when(i == 0)
    def _():
        g = g_ref[...]
        eht = jax.lax.dot_general(vc_ref[...], leT_ref[...],
                                  (((0,), (0,)), ((), ())),
                                  preferred_element_type=jnp.float32)
        eht = eht + vbc_ref[...]
        en2 = _dot(g, eht * eht)
        ehn = eht / jnp.maximum(jnp.sqrt(en2), 1e-12)
        m_ref[...] = jnp.concatenate([ehn] * R, axis=1) * g
        accp_ref[...] = jnp.zeros_like(accp_ref)
        accm_ref[...] = jnp.zeros_like(accm_ref)

    gw = gw_ref[...]
    lgT = (jax.lax.dot_general(gw[:, :D], x1, (((1,), (1,)), ((), ())),
                               preferred_element_type=jnp.float32)
           + jax.lax.dot_general(gw[:, D:], x2, (((1,), (1,)), ((), ())),
                                 preferred_element_type=jnp.float32)
           + gbc_ref[...])

    riT = jax.lax.broadcasted_iota(jnp.int32, lgT.shape, 0)
    m1 = jnp.max(lgT, axis=0, keepdims=True)
    i1 = jnp.min(jnp.where(lgT == m1, riT, R), axis=0, keepdims=True)
    lgm = jnp.where(riT == i1, _NEG, lgT)
    m2 = jnp.max(lgm, axis=0, keepdims=True)
    i2 = jnp.min(jnp.where(lgm == m2, riT, R), axis=0, keepdims=True)

    e2 = jnp.exp(m2 - m1)
    w1 = 1.0 / (1.0 + e2)
    w2 = e2 / (1.0 + e2)

    p = jnp.exp(lgT - m1)
    probsT = p / jnp.sum(p, axis=0, keepdims=True)
    maskT = ((riT == i1) | (riT == i2)).astype(jnp.float32)
    accp_ref[...] += jnp.sum(probsT, axis=1, keepdims=True)
    accm_ref[...] += jnp.sum(maskT, axis=1, keepdims=True)
    aux_ref[...] = (R * AUX_COEF / (B * B)) * jnp.sum(
        accp_ref[...] * accm_ref[...], axis=(0, 1), keepdims=True)

    w8T = (jnp.where(riT == i1, w1, 0.0)
           + jnp.where(riT == i2, w2, 0.0))
    w8 = w8T.T

    xh = _dot(x1, uc_ref[:D]) + _dot(x2, uc_ref[D:]) + ub_ref[...]
    n2 = _dot(xh * xh, g_ref[...])
    xhn = xh / jnp.maximum(jnp.sqrt(n2), 1e-12)

    s = _dot(xhn, m_ref[...])
    es = jnp.exp(s)
    z8 = _dot(es, fz_ref[...])
    wz8 = w8 / z8
    wzl = _dot(wz8, ebc_ref[...])
    llm_probs = _dot(es * wzl, f_ref[...])

    csum = _dot(llm_probs, tri_ref[...])
    rand = rand_ref[...]
    cf = _dot((csum <= rand).astype(jnp.float32), ones_ref[...])
    cnt = cf.astype(jnp.int32)
    sel = jnp.where(cnt == NL, 0, cnt)
    sel_ref[...] = sel

    n_iota = jax.lax.broadcasted_iota(jnp.int32, llm_probs.shape, 1)
    psel = _dot(jnp.where(n_iota == sel, llm_probs, 0.0), ones_ref[...])
    logp_ref[...] = jnp.log(psel)


@jax.jit
def kernel(enhanced_posts_embeddings, selected_reasoning_embeddings,
           llm_embeddings, gate_W, gate_b, U_W, U_b, V_W, V_b):
    uc = U_W.transpose(2, 0, 1).reshape(2 * D, RH)
    ub = U_b.reshape(1, RH)
    vc = V_W.transpose(2, 0, 1).reshape(D, RH)
    vbc = V_b.reshape(RH, 1)
    gbc = gate_b.reshape(R, 1)
    leT = llm_embeddings.T
    rand = jax.random.uniform(jax.random.key(42), (B, 1))

    ri = jnp.arange(RH)
    g_blk = (ri[:, None] // H == ri[None, :] // H).astype(jnp.float32)
    fz = (ri[:, None] // H == jnp.arange(R)[None, :]).astype(jnp.float32)
    ebc = fz.T
    nn = jnp.arange(NL)
    tri = (nn[:, None] <= nn[None, :]).astype(jnp.float32)
    f_fold = (ri[:, None] % NL == nn[None, :]).astype(jnp.float32)
    ones_col = jnp.ones((NL, 1), jnp.float32)

    cspec = lambda shape: pl.BlockSpec(shape, lambda i: (0,) * len(shape))
    sel, logp, aux = pl.pallas_call(
        _moe_kernel,
        grid=(GRID,),
        in_specs=[
            pl.BlockSpec((TILE, D), lambda i: (i, 0)),
            pl.BlockSpec((TILE, D), lambda i: (i, 0)),
            cspec((D, NL)),
            cspec((R, 2 * D)),
            cspec((R, 1)),
            cspec((2 * D, RH)),
            cspec((1, RH)),
            cspec((D, RH)),
            cspec((RH, 1)),
            cspec((RH, RH)),
            cspec((RH, R)),
            cspec((R, RH)),
            cspec((NL, NL)),
            cspec((RH, NL)),
            cspec((NL, 1)),
            pl.BlockSpec((TILE, 1), lambda i: (i, 0)),
        ],
        out_specs=[
            pl.BlockSpec((TILE, 1), lambda i: (i, 0)),
            pl.BlockSpec((TILE, 1), lambda i: (i, 0)),
            pl.BlockSpec((1, 1), lambda i: (0, 0)),
        ],
        out_shape=[
            jax.ShapeDtypeStruct((B, 1), jnp.int32),
            jax.ShapeDtypeStruct((B, 1), jnp.float32),
            jax.ShapeDtypeStruct((1, 1), jnp.float32),
        ],
        scratch_shapes=[pltpu.VMEM((RH, RH), jnp.float32),
                        pltpu.VMEM((R, 1), jnp.float32),
                        pltpu.VMEM((R, 1), jnp.float32)],
    )(enhanced_posts_embeddings, selected_reasoning_embeddings,
      leT, gate_W, gbc, uc, ub, vc, vbc, g_blk, fz, ebc, tri, f_fold,
      ones_col, rand)
    return sel[:, 0], logp, aux[0, 0]
